# baseline (device time: 366249 ns/iter reference)
import jax
import jax.numpy as jnp
from jax import lax
from jax.experimental import pallas as pl
from jax.experimental.pallas import tpu as pltpu

N_DEV = 4


def _allreduce_body(x_ref, out_ref, comm_ref, send_sems, recv_sems):
    my = lax.axis_index("i")
    left = (my - 1) % N_DEV
    right = (my + 1) % N_DEV

    barrier = pltpu.get_barrier_semaphore()
    for nbr in (left, right):
        pl.semaphore_signal(
            barrier, inc=1, device_id=(nbr,), device_id_type=pl.DeviceIdType.MESH
        )
    pl.semaphore_wait(barrier, 2)

    out_ref[...] = x_ref[...]

    rows = out_ref.shape[0] // N_DEV

    def chunk(c):
        return pl.ds(c * rows, rows)

    for h in range(N_DEV - 1):
        send_c = (my - h) % N_DEV
        recv_c = (my - h - 1) % N_DEV
        slot = h % 2
        rdma = pltpu.make_async_remote_copy(
            src_ref=out_ref.at[chunk(send_c)],
            dst_ref=comm_ref.at[slot],
            send_sem=send_sems.at[h],
            recv_sem=recv_sems.at[h],
            device_id=(right,),
            device_id_type=pl.DeviceIdType.MESH,
        )
        rdma.start()
        rdma.wait()
        out_ref[chunk(recv_c), :] += comm_ref[slot]

    for h in range(N_DEV - 1):
        send_c = (my + 1 - h) % N_DEV
        rdma = pltpu.make_async_remote_copy(
            src_ref=out_ref.at[chunk(send_c)],
            dst_ref=out_ref.at[chunk(send_c)],
            send_sem=send_sems.at[N_DEV - 1 + h],
            recv_sem=recv_sems.at[N_DEV - 1 + h],
            device_id=(right,),
            device_id_type=pl.DeviceIdType.MESH,
        )
        rdma.start()
        rdma.wait()


def kernel(x, k, Wp):
    B, S, _ = x.shape
    P = Wp.shape[1]

    pad = jnp.pad(x, ((0, 0), (k.shape[0] - 1, 0), (0, 0)))
    conv = pad[:, 0:S, :] * k[0]
    for t in range(1, k.shape[0]):
        conv = conv + pad[:, t : t + S, :] * k[t]
    a = conv / (1.0 + jnp.exp(-conv))

    a16 = a.astype(jnp.bfloat16).reshape(B * S, -1)
    partial = jnp.dot(
        a16, Wp.astype(jnp.bfloat16), preferred_element_type=jnp.bfloat16
    )

    out = pl.pallas_call(
        _allreduce_body,
        out_shape=jax.ShapeDtypeStruct((B * S, P), jnp.bfloat16),
        in_specs=[pl.BlockSpec(memory_space=pltpu.VMEM)],
        out_specs=pl.BlockSpec(memory_space=pltpu.VMEM),
        scratch_shapes=[
            pltpu.VMEM((2, (B * S) // N_DEV, P), jnp.bfloat16),
            pltpu.SemaphoreType.DMA((2 * (N_DEV - 1),)),
            pltpu.SemaphoreType.DMA((2 * (N_DEV - 1),)),
        ],
        compiler_params=pltpu.CompilerParams(collective_id=0),
    )(partial)
    return out.reshape(B, S, P)


# device time: 222153 ns/iter; 1.6486x vs baseline; 1.6486x over previous
import jax
import jax.numpy as jnp
from jax import lax
from jax.experimental import pallas as pl
from jax.experimental.pallas import tpu as pltpu

N_DEV = 4


def _allreduce_body(x_ref, out_ref, comm_ref, send_sems, recv_sems):
    my = lax.axis_index("i")
    left = (my - 1) % N_DEV
    right = (my + 1) % N_DEV

    barrier = pltpu.get_barrier_semaphore()
    for nbr in (left, right):
        pl.semaphore_signal(
            barrier, inc=1, device_id=(nbr,), device_id_type=pl.DeviceIdType.MESH
        )
    pl.semaphore_wait(barrier, 2)

    out_ref[...] = x_ref[...]

    half = out_ref.shape[0] // 2
    rows = half // N_DEV

    def chunk_a(c):
        return pl.ds(c * rows, rows)

    def chunk_b(c):
        return pl.ds(half + c * rows, rows)

    def rdma(src_rows, dst_ref, dst_idx, sem_i, to):
        return pltpu.make_async_remote_copy(
            src_ref=out_ref.at[src_rows],
            dst_ref=dst_ref.at[dst_idx] if dst_idx is not None else dst_ref,
            send_sem=send_sems.at[sem_i],
            recv_sem=recv_sems.at[sem_i],
            device_id=(to,),
            device_id_type=pl.DeviceIdType.MESH,
        )

    for h in range(N_DEV - 1):
        slot = h % 2
        ra = rdma(chunk_a((my - h) % N_DEV), comm_ref, (0, slot), h, right)
        rb = rdma(chunk_b((my + h) % N_DEV), comm_ref, (1, slot), 6 + h, left)
        ra.start()
        rb.start()
        ra.wait()
        out_ref[chunk_a((my - h - 1) % N_DEV), :] += comm_ref[0, slot]
        rb.wait()
        out_ref[chunk_b((my + h + 1) % N_DEV), :] += comm_ref[1, slot]

    for h in range(N_DEV - 1):
        sa = chunk_a((my + 1 - h) % N_DEV)
        sb = chunk_b((my - 1 + h) % N_DEV)
        ga = pltpu.make_async_remote_copy(
            src_ref=out_ref.at[sa], dst_ref=out_ref.at[sa],
            send_sem=send_sems.at[3 + h], recv_sem=recv_sems.at[3 + h],
            device_id=(right,), device_id_type=pl.DeviceIdType.MESH,
        )
        gb = pltpu.make_async_remote_copy(
            src_ref=out_ref.at[sb], dst_ref=out_ref.at[sb],
            send_sem=send_sems.at[9 + h], recv_sem=recv_sems.at[9 + h],
            device_id=(left,), device_id_type=pl.DeviceIdType.MESH,
        )
        ga.start()
        gb.start()
        ga.wait()
        gb.wait()


def kernel(x, k, Wp):
    B, S, _ = x.shape
    P = Wp.shape[1]

    x16 = x.astype(jnp.bfloat16)
    k16 = k.astype(jnp.bfloat16)
    pad = jnp.pad(x16, ((0, 0), (k.shape[0] - 1, 0), (0, 0)))
    conv = pad[:, 0:S, :] * k16[0]
    for t in range(1, k.shape[0]):
        conv = conv + pad[:, t : t + S, :] * k16[t]
    a = conv * jax.nn.sigmoid(conv)

    partial = jnp.dot(
        a.reshape(B * S, -1),
        Wp.astype(jnp.bfloat16),
        preferred_element_type=jnp.bfloat16,
    )

    out = pl.pallas_call(
        _allreduce_body,
        out_shape=jax.ShapeDtypeStruct((B * S, P), jnp.bfloat16),
        in_specs=[pl.BlockSpec(memory_space=pltpu.VMEM)],
        out_specs=pl.BlockSpec(memory_space=pltpu.VMEM),
        scratch_shapes=[
            pltpu.VMEM((2, 2, (B * S) // (2 * N_DEV), P), jnp.bfloat16),
            pltpu.SemaphoreType.DMA((12,)),
            pltpu.SemaphoreType.DMA((12,)),
        ],
        compiler_params=pltpu.CompilerParams(collective_id=0),
    )(partial)
    return out.reshape(B, S, P)


# device time: 190148 ns/iter; 1.9261x vs baseline; 1.1683x over previous
import jax
import jax.numpy as jnp
from jax import lax
from jax.experimental import pallas as pl
from jax.experimental.pallas import tpu as pltpu

N_DEV = 4
CH = 1024
HALF = N_DEV * CH
TAPS = 4


def _body(x_ref, k_ref, w_ref, out_ref, comm_ref, send_sems, recv_sems):
    my = lax.axis_index("i")
    left = (my - 1) % N_DEV
    right = (my + 1) % N_DEV

    barrier = pltpu.get_barrier_semaphore()
    for nbr in (left, right):
        pl.semaphore_signal(
            barrier, inc=1, device_id=(nbr,), device_id_type=pl.DeviceIdType.MESH
        )
    pl.semaphore_wait(barrier, 2)

    def compute_chunk(r0):
        xc = x_ref[pl.ds(r0, CH), :]
        hl8 = x_ref[pl.ds(pl.multiple_of(jnp.maximum(r0 - 8, 0), 8), 8), :]
        hl = hl8[8 - (TAPS - 1) :]
        first = lax.rem(r0, HALF) == 0
        hl = jnp.where(first, jnp.zeros_like(hl), hl)
        xp = jnp.concatenate([hl, xc], axis=0)
        conv = xp[TAPS - 1 : TAPS - 1 + CH] * k_ref[TAPS - 1 : TAPS, :]
        for t in range(TAPS - 1):
            conv = conv + xp[t : t + CH] * k_ref[t : t + 1, :]
        a = conv * jax.nn.sigmoid(conv)
        out_ref[pl.ds(r0, CH), :] = jnp.dot(
            a, w_ref[...], preferred_element_type=jnp.float32
        ).astype(jnp.bfloat16)

    def row_a(c):
        return pl.multiple_of((c % N_DEV) * CH, CH)

    def row_b(c):
        return pl.multiple_of(HALF + (c % N_DEV) * CH, CH)

    def rdma(rows, dst, sem_i, to):
        return pltpu.make_async_remote_copy(
            src_ref=out_ref.at[pl.ds(rows, CH)],
            dst_ref=dst,
            send_sem=send_sems.at[sem_i],
            recv_sem=recv_sems.at[sem_i],
            device_id=(to,),
            device_id_type=pl.DeviceIdType.MESH,
        )

    compute_chunk(row_a(my))
    compute_chunk(row_b(my))

    for h in range(N_DEV - 1):
        slot = h % 2
        ra = rdma(row_a(my - h), comm_ref.at[0, slot], h, right)
        rb = rdma(row_b(my + h), comm_ref.at[1, slot], 6 + h, left)
        ra.start()
        rb.start()
        compute_chunk(row_a(my - h - 1))
        compute_chunk(row_b(my + h + 1))
        ra.wait()
        out_ref[pl.ds(row_a(my - h - 1), CH), :] += comm_ref[0, slot]
        rb.wait()
        out_ref[pl.ds(row_b(my + h + 1), CH), :] += comm_ref[1, slot]

    for h in range(N_DEV - 1):
        sa = row_a(my + 1 - h)
        sb = row_b(my - 1 + h)
        ga = rdma(sa, out_ref.at[pl.ds(sa, CH)], 3 + h, right)
        gb = rdma(sb, out_ref.at[pl.ds(sb, CH)], 9 + h, left)
        ga.start()
        gb.start()
        ga.wait()
        gb.wait()


def kernel(x, k, Wp):
    B, S, _ = x.shape
    P = Wp.shape[1]

    out = pl.pallas_call(
        _body,
        out_shape=jax.ShapeDtypeStruct((B * S, P), jnp.bfloat16),
        in_specs=[pl.BlockSpec(memory_space=pltpu.VMEM)] * 3,
        out_specs=pl.BlockSpec(memory_space=pltpu.VMEM),
        scratch_shapes=[
            pltpu.VMEM((2, 2, CH, P), jnp.bfloat16),
            pltpu.SemaphoreType.DMA((12,)),
            pltpu.SemaphoreType.DMA((12,)),
        ],
        compiler_params=pltpu.CompilerParams(collective_id=0),
    )(
        x.astype(jnp.bfloat16).reshape(B * S, -1),
        k.astype(jnp.bfloat16),
        Wp.astype(jnp.bfloat16),
    )
    return out.reshape(B, S, P)


# device time: 170567 ns/iter; 2.1472x vs baseline; 1.1148x over previous
import jax
import jax.numpy as jnp
from jax import lax
from jax.experimental import pallas as pl
from jax.experimental.pallas import tpu as pltpu

N_DEV = 4
CH = 1024
HALF = N_DEV * CH
TAPS = 4
NCHUNK = 2 * N_DEV


def _body(x_hbm, k_ref, w_ref, out_ref, xbuf, hbuf, comm_ref,
          xsems, hsems, send_sems, recv_sems):
    my = lax.axis_index("i")
    left = (my - 1) % N_DEV
    right = (my + 1) % N_DEV

    barrier = pltpu.get_barrier_semaphore()
    for nbr in (left, right):
        pl.semaphore_signal(
            barrier, inc=1, device_id=(nbr,), device_id_type=pl.DeviceIdType.MESH
        )
    pl.semaphore_wait(barrier, 2)

    def row_a(c):
        return pl.multiple_of((c % N_DEV) * CH, CH)

    def row_b(c):
        return pl.multiple_of(HALF + (c % N_DEV) * CH, CH)

    sched = []
    for h in range(N_DEV):
        sched.append(row_a(my - h))
        sched.append(row_b(my + h))

    def fetch(i):
        r0 = sched[i]
        s = i % 2
        cp = pltpu.make_async_copy(
            x_hbm.at[pl.ds(r0, CH), :], xbuf.at[s], xsems.at[i]
        )
        hp = pltpu.make_async_copy(
            x_hbm.at[pl.ds(pl.multiple_of(jnp.maximum(r0 - 8, 0), 8), 8), :],
            hbuf.at[s],
            hsems.at[i],
        )
        cp.start()
        hp.start()
        return cp, hp

    def compute(i):
        r0 = sched[i]
        s = i % 2
        pltpu.make_async_copy(x_hbm.at[pl.ds(r0, CH), :], xbuf.at[s],
                              xsems.at[i]).wait()
        pltpu.make_async_copy(x_hbm.at[pl.ds(0, 8), :], hbuf.at[s],
                              hsems.at[i]).wait()
        xc = xbuf[s]
        hl = hbuf[s][8 - (TAPS - 1):]
        first = lax.rem(r0, HALF) == 0
        hl = jnp.where(first, jnp.zeros_like(hl), hl)
        xp = jnp.concatenate([hl, xc], axis=0)
        conv = xp[TAPS - 1: TAPS - 1 + CH] * k_ref[TAPS - 1: TAPS, :]
        for t in range(TAPS - 1):
            conv = conv + xp[t: t + CH] * k_ref[t: t + 1, :]
        a = (conv * jax.nn.sigmoid(conv)).astype(jnp.bfloat16)
        out_ref[pl.ds(r0, CH), :] = jnp.dot(
            a, w_ref[...], preferred_element_type=jnp.float32
        ).astype(jnp.bfloat16)

    def rdma(rows, dst, sem_i, to):
        return pltpu.make_async_remote_copy(
            src_ref=out_ref.at[pl.ds(rows, CH)],
            dst_ref=dst,
            send_sem=send_sems.at[sem_i],
            recv_sem=recv_sems.at[sem_i],
            device_id=(to,),
            device_id_type=pl.DeviceIdType.MESH,
        )

    fetch(0)
    fetch(1)
    compute(0)
    fetch(2)
    compute(1)
    fetch(3)

    for h in range(N_DEV - 1):
        slot = h % 2
        ra = rdma(row_a(my - h), comm_ref.at[0, slot], h, right)
        rb = rdma(row_b(my + h), comm_ref.at[1, slot], 6 + h, left)
        ra.start()
        rb.start()
        compute(2 * h + 2)
        if 2 * h + 4 < NCHUNK:
            fetch(2 * h + 4)
        compute(2 * h + 3)
        if 2 * h + 5 < NCHUNK:
            fetch(2 * h + 5)
        ra.wait()
        out_ref[pl.ds(row_a(my - h - 1), CH), :] += comm_ref[0, slot]
        rb.wait()
        out_ref[pl.ds(row_b(my + h + 1), CH), :] += comm_ref[1, slot]

    for h in range(N_DEV - 1):
        sa = row_a(my + 1 - h)
        sb = row_b(my - 1 + h)
        ga = rdma(sa, out_ref.at[pl.ds(sa, CH)], 3 + h, right)
        gb = rdma(sb, out_ref.at[pl.ds(sb, CH)], 9 + h, left)
        ga.start()
        gb.start()
        ga.wait()
        gb.wait()


def kernel(x, k, Wp):
    B, S, _ = x.shape
    P = Wp.shape[1]

    out = pl.pallas_call(
        _body,
        out_shape=jax.ShapeDtypeStruct((B * S, P), jnp.bfloat16),
        in_specs=[
            pl.BlockSpec(memory_space=pl.ANY),
            pl.BlockSpec(memory_space=pltpu.VMEM),
            pl.BlockSpec(memory_space=pltpu.VMEM),
        ],
        out_specs=pl.BlockSpec(memory_space=pltpu.VMEM),
        scratch_shapes=[
            pltpu.VMEM((2, CH, P), jnp.float32),
            pltpu.VMEM((2, 8, P), jnp.float32),
            pltpu.VMEM((2, 2, CH, P), jnp.bfloat16),
            pltpu.SemaphoreType.DMA((NCHUNK,)),
            pltpu.SemaphoreType.DMA((NCHUNK,)),
            pltpu.SemaphoreType.DMA((12,)),
            pltpu.SemaphoreType.DMA((12,)),
        ],
        compiler_params=pltpu.CompilerParams(collective_id=0),
    )(
        x.reshape(B * S, -1),
        k,
        Wp.astype(jnp.bfloat16),
    )
    return out.reshape(B, S, P)


# device time: 163065 ns/iter; 2.2460x vs baseline; 1.0460x over previous
import jax
import jax.numpy as jnp
from jax import lax
from jax.experimental import pallas as pl
from jax.experimental.pallas import tpu as pltpu

N_DEV = 4
CH = 1024
SUB = CH // 2
HALF = N_DEV * CH
TAPS = 4
NCHUNK = 2 * N_DEV
RINGS = ((0, 0), (0, 1), (1, 0), (1, 1))


def _body(x_hbm, k_ref, w_ref, out_ref, xbuf, hbuf, comm_ref,
          xsems, hsems, send_sems, recv_sems):
    my = lax.axis_index("i")
    left = (my - 1) % N_DEV
    right = (my + 1) % N_DEV

    barrier = pltpu.get_barrier_semaphore()
    for nbr in (left, right):
        pl.semaphore_signal(
            barrier, inc=1, device_id=(nbr,), device_id_type=pl.DeviceIdType.MESH
        )
    pl.semaphore_wait(barrier, 2)

    def chunk_row(d, c):
        return pl.multiple_of((d * HALF + (c % N_DEV) * CH) % (2 * HALF), CH)

    def sub_row(r, c):
        d, s = RINGS[r]
        return pl.multiple_of(chunk_row(d, c) + s * SUB, SUB)

    def rs_send_chunk(r, h):
        return my - h if RINGS[r][0] == 0 else my + h

    def ag_send_chunk(r, h):
        return my + 1 - h if RINGS[r][0] == 0 else my - 1 + h

    def peer(r):
        return right if RINGS[r][0] == 0 else left

    sched = [chunk_row(h % 2, my - h // 2 if h % 2 == 0 else my + h // 2)
             for h in range(NCHUNK)]

    def fetch(i):
        r0 = sched[i]
        s = i % 2
        pltpu.make_async_copy(
            x_hbm.at[pl.ds(r0, CH), :], xbuf.at[s], xsems.at[i]
        ).start()
        pltpu.make_async_copy(
            x_hbm.at[pl.ds(pl.multiple_of(jnp.maximum(r0 - 8, 0), 8), 8), :],
            hbuf.at[s],
            hsems.at[i],
        ).start()

    def compute(i):
        r0 = sched[i]
        s = i % 2
        pltpu.make_async_copy(x_hbm.at[pl.ds(r0, CH), :], xbuf.at[s],
                              xsems.at[i]).wait()
        pltpu.make_async_copy(x_hbm.at[pl.ds(0, 8), :], hbuf.at[s],
                              hsems.at[i]).wait()
        xc = xbuf[s]
        hl = hbuf[s][8 - (TAPS - 1):]
        first = lax.rem(r0, HALF) == 0
        hl = jnp.where(first, jnp.zeros_like(hl), hl)
        xp = jnp.concatenate([hl, xc], axis=0)
        conv = xp[TAPS - 1: TAPS - 1 + CH] * k_ref[TAPS - 1: TAPS, :]
        for t in range(TAPS - 1):
            conv = conv + xp[t: t + CH] * k_ref[t: t + 1, :]
        a = (conv * jax.nn.sigmoid(conv)).astype(jnp.bfloat16)
        out_ref[pl.ds(r0, CH), :] = jnp.dot(
            a, w_ref[...], preferred_element_type=jnp.float32
        ).astype(jnp.bfloat16)

    def rs_rdma(r, h):
        d, sub = RINGS[r]
        return pltpu.make_async_remote_copy(
            src_ref=out_ref.at[pl.ds(sub_row(r, rs_send_chunk(r, h)), SUB)],
            dst_ref=comm_ref.at[d, sub, h % 2],
            send_sem=send_sems.at[r * 6 + h],
            recv_sem=recv_sems.at[r * 6 + h],
            device_id=(peer(r),),
            device_id_type=pl.DeviceIdType.MESH,
        )

    def rs_acc(r, h):
        d, sub = RINGS[r]
        rows = sub_row(r, rs_send_chunk(r, h) + (-1 if d == 0 else 1))
        out_ref[pl.ds(rows, SUB), :] += comm_ref[d, sub, h % 2]

    def ag_rdma(r, h):
        rows = sub_row(r, ag_send_chunk(r, h))
        return pltpu.make_async_remote_copy(
            src_ref=out_ref.at[pl.ds(rows, SUB)],
            dst_ref=out_ref.at[pl.ds(rows, SUB)],
            send_sem=send_sems.at[r * 6 + 3 + h],
            recv_sem=recv_sems.at[r * 6 + 3 + h],
            device_id=(peer(r),),
            device_id_type=pl.DeviceIdType.MESH,
        )

    fetch(0)
    fetch(1)
    compute(0)
    fetch(2)
    compute(1)
    fetch(3)

    rs = {r: rs_rdma(r, 0) for r in range(4)}
    for r in range(4):
        rs[r].start()
    for h in range(N_DEV - 1):
        compute(2 * h + 2)
        if 2 * h + 4 < NCHUNK:
            fetch(2 * h + 4)
        compute(2 * h + 3)
        if 2 * h + 5 < NCHUNK:
            fetch(2 * h + 5)
        for r in range(4):
            rs[r].wait()
            rs_acc(r, h)
            if h < N_DEV - 2:
                rs[r] = rs_rdma(r, h + 1)
                rs[r].start()
            else:
                rs[r] = ag_rdma(r, 0)
                rs[r].start()

    ag = rs
    for h in range(N_DEV - 1):
        for r in range(4):
            ag[r].wait()
            if h < N_DEV - 2:
                ag[r] = ag_rdma(r, h + 1)
                ag[r].start()


def kernel(x, k, Wp):
    B, S, _ = x.shape
    P = Wp.shape[1]

    out = pl.pallas_call(
        _body,
        out_shape=jax.ShapeDtypeStruct((B * S, P), jnp.bfloat16),
        in_specs=[
            pl.BlockSpec(memory_space=pl.ANY),
            pl.BlockSpec(memory_space=pltpu.VMEM),
            pl.BlockSpec(memory_space=pltpu.VMEM),
        ],
        out_specs=pl.BlockSpec(memory_space=pltpu.VMEM),
        scratch_shapes=[
            pltpu.VMEM((2, CH, P), jnp.float32),
            pltpu.VMEM((2, 8, P), jnp.float32),
            pltpu.VMEM((2, 2, 2, SUB, P), jnp.bfloat16),
            pltpu.SemaphoreType.DMA((NCHUNK,)),
            pltpu.SemaphoreType.DMA((NCHUNK,)),
            pltpu.SemaphoreType.DMA((24,)),
            pltpu.SemaphoreType.DMA((24,)),
        ],
        compiler_params=pltpu.CompilerParams(collective_id=0),
    )(
        x.reshape(B * S, -1),
        k,
        Wp.astype(jnp.bfloat16),
    )
    return out.reshape(B, S, P)


# device time: 156818 ns/iter; 2.3355x vs baseline; 1.0398x over previous
import jax
import jax.numpy as jnp
from jax import lax
from jax.experimental import pallas as pl
from jax.experimental.pallas import tpu as pltpu

N_DEV = 4
CH = 1024
SUB = CH // 2
HALF = N_DEV * CH
TAPS = 4
NCHUNK = 2 * N_DEV
RINGS = ((0, 0), (0, 1), (1, 0), (1, 1))


def _body(x_hbm, k_ref, w_ref, out_ref, xbuf, hbuf, comm_ref,
          xsems, hsems, send_sems, recv_sems):
    my = lax.axis_index("i")
    left = (my - 1) % N_DEV
    right = (my + 1) % N_DEV

    barrier = pltpu.get_barrier_semaphore()
    for nbr in (left, right):
        pl.semaphore_signal(
            barrier, inc=1, device_id=(nbr,), device_id_type=pl.DeviceIdType.MESH
        )
    pl.semaphore_wait(barrier, 2)

    def chunk_row(d, c):
        return pl.multiple_of((d * HALF + (c % N_DEV) * CH) % (2 * HALF), CH)

    def sub_row(r, c):
        d, s = RINGS[r]
        return pl.multiple_of(chunk_row(d, c) + s * SUB, SUB)

    def rs_send_chunk(r, h):
        return my - h if RINGS[r][0] == 0 else my + h

    def ag_send_chunk(r, h):
        return my + 1 - h if RINGS[r][0] == 0 else my - 1 + h

    def peer(r):
        return right if RINGS[r][0] == 0 else left

    sched = [chunk_row(h % 2, my - h // 2 if h % 2 == 0 else my + h // 2)
             for h in range(NCHUNK)]

    def fetch(i):
        r0 = sched[i]
        s = i % 2
        pltpu.make_async_copy(
            x_hbm.at[pl.ds(r0, CH), :], xbuf.at[s], xsems.at[i]
        ).start()
        pltpu.make_async_copy(
            x_hbm.at[pl.ds(pl.multiple_of(jnp.maximum(r0 - 8, 0), 8), 8), :],
            hbuf.at[s],
            hsems.at[i],
        ).start()

    def compute(i):
        r0 = sched[i]
        s = i % 2
        pltpu.make_async_copy(x_hbm.at[pl.ds(r0, CH), :], xbuf.at[s],
                              xsems.at[i]).wait()
        pltpu.make_async_copy(x_hbm.at[pl.ds(0, 8), :], hbuf.at[s],
                              hsems.at[i]).wait()
        xc = xbuf[s]
        hl = hbuf[s][8 - (TAPS - 1):]
        first = lax.rem(r0, HALF) == 0
        hl = jnp.where(first, jnp.zeros_like(hl), hl)
        xp = jnp.concatenate([hl, xc], axis=0)
        conv = xp[TAPS - 1: TAPS - 1 + CH] * k_ref[TAPS - 1: TAPS, :]
        for t in range(TAPS - 1):
            conv = conv + xp[t: t + CH] * k_ref[t: t + 1, :]
        a = (conv * jax.nn.sigmoid(conv)).astype(jnp.bfloat16)
        out_ref[pl.ds(r0, CH), :] = jnp.dot(
            a, w_ref[...], preferred_element_type=jnp.float32
        ).astype(jnp.bfloat16)

    def compute_half(i, half):
        r0 = sched[i]
        s = i % 2
        if half == 0:
            pltpu.make_async_copy(x_hbm.at[pl.ds(r0, CH), :], xbuf.at[s],
                                  xsems.at[i]).wait()
            pltpu.make_async_copy(x_hbm.at[pl.ds(0, 8), :], hbuf.at[s],
                                  hsems.at[i]).wait()
        xc = xbuf[s][half * SUB: half * SUB + SUB]
        if half == 0:
            hl = hbuf[s][8 - (TAPS - 1):]
            first = lax.rem(r0, HALF) == 0
            hl = jnp.where(first, jnp.zeros_like(hl), hl)
        else:
            hl = xbuf[s][SUB - (TAPS - 1): SUB]
        xp = jnp.concatenate([hl, xc], axis=0)
        conv = xp[TAPS - 1: TAPS - 1 + SUB] * k_ref[TAPS - 1: TAPS, :]
        for t in range(TAPS - 1):
            conv = conv + xp[t: t + SUB] * k_ref[t: t + 1, :]
        a = (conv * jax.nn.sigmoid(conv)).astype(jnp.bfloat16)
        out_ref[pl.ds(pl.multiple_of(r0 + half * SUB, SUB), SUB), :] = jnp.dot(
            a, w_ref[...], preferred_element_type=jnp.float32
        ).astype(jnp.bfloat16)

    def rs_rdma(r, h):
        d, sub = RINGS[r]
        return pltpu.make_async_remote_copy(
            src_ref=out_ref.at[pl.ds(sub_row(r, rs_send_chunk(r, h)), SUB)],
            dst_ref=comm_ref.at[d, sub, h % 2],
            send_sem=send_sems.at[r * 6 + h],
            recv_sem=recv_sems.at[r * 6 + h],
            device_id=(peer(r),),
            device_id_type=pl.DeviceIdType.MESH,
        )

    def rs_acc(r, h):
        d, sub = RINGS[r]
        rows = sub_row(r, rs_send_chunk(r, h) + (-1 if d == 0 else 1))
        out_ref[pl.ds(rows, SUB), :] += comm_ref[d, sub, h % 2]

    def ag_rdma(r, h):
        rows = sub_row(r, ag_send_chunk(r, h))
        return pltpu.make_async_remote_copy(
            src_ref=out_ref.at[pl.ds(rows, SUB)],
            dst_ref=out_ref.at[pl.ds(rows, SUB)],
            send_sem=send_sems.at[r * 6 + 3 + h],
            recv_sem=recv_sems.at[r * 6 + 3 + h],
            device_id=(peer(r),),
            device_id_type=pl.DeviceIdType.MESH,
        )

    rs = {}
    fetch(0)
    fetch(1)
    for i, half, r in ((0, 0, 0), (1, 0, 2), (0, 1, 1), (1, 1, 3)):
        compute_half(i, half)
        rs[r] = rs_rdma(r, 0)
        rs[r].start()
        if half == 1:
            fetch(2 + i)

    for h in range(N_DEV - 1):
        compute(2 * h + 2)
        if 2 * h + 4 < NCHUNK:
            fetch(2 * h + 4)
        compute(2 * h + 3)
        if 2 * h + 5 < NCHUNK:
            fetch(2 * h + 5)
        for r in (0, 2, 1, 3):
            rs[r].wait()
            rs_acc(r, h)
            if h < N_DEV - 2:
                rs[r] = rs_rdma(r, h + 1)
                rs[r].start()
            else:
                rs[r] = ag_rdma(r, 0)
                rs[r].start()

    ag = rs
    for h in range(N_DEV - 1):
        for r in (0, 2, 1, 3):
            ag[r].wait()
            if h < N_DEV - 2:
                ag[r] = ag_rdma(r, h + 1)
                ag[r].start()


def kernel(x, k, Wp):
    B, S, _ = x.shape
    P = Wp.shape[1]

    out = pl.pallas_call(
        _body,
        out_shape=jax.ShapeDtypeStruct((B * S, P), jnp.bfloat16),
        in_specs=[
            pl.BlockSpec(memory_space=pl.ANY),
            pl.BlockSpec(memory_space=pltpu.VMEM),
            pl.BlockSpec(memory_space=pltpu.VMEM),
        ],
        out_specs=pl.BlockSpec(memory_space=pltpu.VMEM),
        scratch_shapes=[
            pltpu.VMEM((2, CH, P), jnp.float32),
            pltpu.VMEM((2, 8, P), jnp.float32),
            pltpu.VMEM((2, 2, 2, SUB, P), jnp.bfloat16),
            pltpu.SemaphoreType.DMA((NCHUNK,)),
            pltpu.SemaphoreType.DMA((NCHUNK,)),
            pltpu.SemaphoreType.DMA((24,)),
            pltpu.SemaphoreType.DMA((24,)),
        ],
        compiler_params=pltpu.CompilerParams(collective_id=0),
    )(
        x.reshape(B * S, -1),
        k,
        Wp.astype(jnp.bfloat16),
    )
    return out.reshape(B, S, P)
